# baseline (speedup 1.0000x reference)
import functools
import math

import jax
import jax.numpy as jnp
from jax import lax
from jax.experimental import pallas as pl
from jax.experimental.pallas import tpu as pltpu
from jax.experimental.pallas import tpu_sc as plsc

N_NODES = 10000
N_EDGES = 320000
N_MOL = 512
CUTOFF = 6.0
NG = 50
NGP = 64
NC = 2
NS = 16
NW = NC * NS

M_TAB = 32768
D_MAX = 6.2
H_TAB = D_MAX / M_TAB

_f32 = jnp.float32


def _ssp(x):
    return jnp.maximum(x, 0.0) + jnp.log(1.0 + jnp.exp(-jnp.abs(x))) - math.log(2.0)


def _mesh():
    return plsc.VectorSubcoreMesh(core_axis_name="c", subcore_axis_name="s")


@functools.lru_cache(maxsize=None)
def _sc_gather(n_rows, d, n_idx, k):
    ew = n_idx // NW
    steps = ew // k
    assert ew % k == 0 and k % 8 == 0 and ew * NW == n_idx

    def body(table_hbm, idx_hbm, out_hbm, idx_v, rows_v, sem):
        wid = lax.axis_index("s") * NC + lax.axis_index("c")
        base = wid * ew

        def step(i, c):
            off = base + i * k
            pltpu.sync_copy(idx_hbm.at[pl.ds(off, k)], idx_v)
            pltpu.async_copy(table_hbm.at[idx_v], rows_v, sem).wait()
            pltpu.sync_copy(rows_v, out_hbm.at[pl.ds(off, k)])
            return c

        lax.fori_loop(0, steps, step, 0)

    return pl.kernel(
        body,
        out_type=jax.ShapeDtypeStruct((n_idx, d), _f32),
        mesh=_mesh(),
        compiler_params=pltpu.CompilerParams(use_tc_tiling_on_sc=False),
        scratch_types=[
            pltpu.VMEM((k,), jnp.int32),
            pltpu.VMEM((k, d), _f32),
            pltpu.SemaphoreType.DMA,
        ],
    )


@functools.lru_cache(maxsize=None)
def _sc_msg_scatter(n_rows, d, n_idx, k):
    ew = n_idx // NW
    steps = ew // k
    zr = n_rows // NS
    zc = 25
    zsteps = zr // zc
    assert ew % k == 0 and k % 8 == 0 and zr * NS == n_rows and zr % zc == 0

    assert steps % 2 == 1 and k % 16 == 0

    def body(tab_hbm, d_hbm, hlin_hbm, src_hbm, dst_hbm, out_hbm,
             sidx0, sidx1, didx0, didx1, dch0, dch1, tidx0, tidx1,
             x0, x1, tr0, tr1, zbuf, accum,
             isem0, isem1, jsem0, jsem1, gsem0, gsem1, wsem0, wsem1):
        sidx = [sidx0, sidx1]
        didx = [didx0, didx1]
        dch = [dch0, dch1]
        tidx = [tidx0, tidx1]
        xrows = [x0, x1]
        trows = [tr0, tr1]
        isem = [isem0, isem1]
        jsem = [jsem0, jsem1]
        gsem = [gsem0, gsem1]
        wsem = [wsem0, wsem1]
        cid = lax.axis_index("c")
        sid = lax.axis_index("s")
        wid = sid * NC + cid
        base = wid * ew
        inv_h = 1.0 / H_TAB

        def zrow(r, c):
            for j in range(d // 16):
                zbuf[r, pl.ds(j * 16, 16)] = jnp.zeros((16,), _f32)
            return c

        lax.fori_loop(0, zc, zrow, 0)
        for j in range(zsteps):
            pltpu.sync_copy(zbuf, accum.at[pl.ds(sid * zr + j * zc, zc)])
        plsc.subcore_barrier()

        def eidx(i, b):
            off = base + i * k
            pltpu.async_copy(src_hbm.at[pl.ds(off, k)], sidx[b], isem[b])
            pltpu.async_copy(dst_hbm.at[pl.ds(off, k)], didx[b], jsem[b])
            pltpu.async_copy(d_hbm.at[pl.ds(off, k)], dch[b], isem[b])

        def emain(i, b):
            off = base + i * k
            pltpu.make_async_copy(src_hbm.at[pl.ds(off, k)], sidx[b],
                                  isem[b]).wait()
            pltpu.make_async_copy(d_hbm.at[pl.ds(off, k)], dch[b],
                                  isem[b]).wait()

            def trow(g, c2):
                sl = pl.ds(g * 16, 16)
                iv = (dch[b][sl] * inv_h + 0.5).astype(jnp.int32)
                tidx[b][sl] = jnp.minimum(iv, M_TAB - 1)
                return c2

            lax.fori_loop(0, k // 16, trow, 0)
            pltpu.async_copy(hlin_hbm.at[sidx[b]], xrows[b], gsem[b])
            pltpu.async_copy(tab_hbm.at[tidx[b]], trows[b], wsem[b])

        def consume(i, b):
            pltpu.make_async_copy(hlin_hbm.at[sidx[b]], xrows[b],
                                  gsem[b]).wait()
            pltpu.make_async_copy(tab_hbm.at[tidx[b]], trows[b],
                                  wsem[b]).wait()
            off = base + i * k
            pltpu.make_async_copy(dst_hbm.at[pl.ds(off, k)], didx[b],
                                  jsem[b]).wait()

            def mrow(r, c2):
                for j in range(d // 16):
                    sl = pl.ds(j * 16, 16)
                    trows[b][r, sl] = trows[b][r, sl] * xrows[b][r, sl]
                return c2

            lax.fori_loop(0, k, mrow, 0)
            pltpu.sync_copy(trows[b], accum.at[didx[b]], add=True)

        eidx(0, 0)
        eidx(1, 1)
        emain(0, 0)

        def pair(p, c):
            i0 = 2 * p
            emain(i0 + 1, 1)
            consume(i0, 0)
            eidx(i0 + 2, 0)
            i1 = i0 + 1
            emain(i1 + 1, 0)
            consume(i1, 1)

            @pl.when(i1 + 2 < steps)
            def _():
                eidx(i1 + 2, 1)

            return c

        lax.fori_loop(0, (steps - 1) // 2, pair, 0)
        consume(steps - 1, 0)
        plsc.subcore_barrier()

        pltpu.sync_copy(accum.at[pl.ds(sid * zr, zr)],
                        out_hbm.at[cid].at[pl.ds(sid * zr, zr)])

    return pl.kernel(
        body,
        out_type=jax.ShapeDtypeStruct((NC, n_rows, d), _f32),
        mesh=_mesh(),
        compiler_params=pltpu.CompilerParams(use_tc_tiling_on_sc=False),
        scratch_types=[
            pltpu.VMEM((k,), jnp.int32),
            pltpu.VMEM((k,), jnp.int32),
            pltpu.VMEM((k,), jnp.int32),
            pltpu.VMEM((k,), jnp.int32),
            pltpu.VMEM((k,), _f32),
            pltpu.VMEM((k,), _f32),
            pltpu.VMEM((k,), jnp.int32),
            pltpu.VMEM((k,), jnp.int32),
            pltpu.VMEM((k, d), _f32),
            pltpu.VMEM((k, d), _f32),
            pltpu.VMEM((k, d), _f32),
            pltpu.VMEM((k, d), _f32),
            pltpu.VMEM((zc, d), _f32),
            pltpu.VMEM_SHARED((n_rows, d), _f32),
            pltpu.SemaphoreType.DMA,
            pltpu.SemaphoreType.DMA,
            pltpu.SemaphoreType.DMA,
            pltpu.SemaphoreType.DMA,
            pltpu.SemaphoreType.DMA,
            pltpu.SemaphoreType.DMA,
            pltpu.SemaphoreType.DMA,
            pltpu.SemaphoreType.DMA,
        ],
    )


@functools.lru_cache(maxsize=None)
def _embed(h):
    r = 1000
    grid = N_NODES // r
    zp = 104

    def body(z_r, emb_r, l1_r, h_r, hlin_r):
        oh = (z_r[...] == lax.broadcasted_iota(jnp.int32, (r, zp), 1)
              ).astype(_f32)
        h0 = jnp.dot(oh, emb_r[...], preferred_element_type=_f32)
        h_r[...] = h0
        hlin_r[...] = jnp.dot(h0, l1_r[...], preferred_element_type=_f32)

    return pl.pallas_call(
        body,
        grid=(grid,),
        in_specs=[
            pl.BlockSpec((r, 1), lambda i: (i, 0)),
            pl.BlockSpec((zp, h), lambda i: (0, 0)),
            pl.BlockSpec((h, h), lambda i: (0, 0)),
        ],
        out_specs=(pl.BlockSpec((r, h), lambda i: (i, 0)),
                   pl.BlockSpec((r, h), lambda i: (i, 0))),
        out_shape=(jax.ShapeDtypeStruct((N_NODES, h), _f32),
                   jax.ShapeDtypeStruct((N_NODES, h), _f32)),
    )


@functools.lru_cache(maxsize=None)
def _edge_d():
    te = 2000
    grid = N_EDGES // te

    def body(px_r, py_r, out_r):
        dxyz = px_r[...] - py_r[...]
        d2 = jnp.sum(dxyz * dxyz, axis=1, keepdims=True)
        out_r[...] = jnp.sqrt(d2 + 1e-12)

    return pl.pallas_call(
        body,
        grid=(grid,),
        in_specs=[pl.BlockSpec((te, 16), lambda i: (i, 0)),
                  pl.BlockSpec((te, 16), lambda i: (i, 0))],
        out_specs=pl.BlockSpec((te, 1), lambda i: (i, 0)),
        out_shape=jax.ShapeDtypeStruct((N_EDGES, 1), _f32),
    )


@functools.lru_cache(maxsize=None)
def _tables(hdims):
    tm = 2048
    grid = M_TAB // tm
    delta = CUTOFF / (NG - 1)
    coeff = -0.5 / delta ** 2
    nb = len(hdims)

    def body(*refs):
        ins = refs[:4 * nb]
        outs = refs[4 * nb:]
        i = pl.program_id(0)
        row = (i * tm + lax.broadcasted_iota(jnp.int32, (tm, 1), 0)
               ).astype(_f32)
        dg = row * H_TAB
        offs = lax.broadcasted_iota(jnp.int32, (tm, NGP), 1).astype(_f32) * delta
        dd = dg - offs
        rbf = jnp.exp(coeff * (dd * dd))
        c = 0.5 * (jnp.cos(dg * (math.pi / CUTOFF)) + 1.0)
        c = jnp.where(dg < CUTOFF, c, 0.0)
        for bi in range(nb):
            w1_r, b1_r, w2_r, b2_r = ins[4 * bi:4 * bi + 4]
            t = _ssp(jnp.dot(rbf, w1_r[...], preferred_element_type=_f32)
                     + b1_r[...])
            w = jnp.dot(t, w2_r[...], preferred_element_type=_f32) + b2_r[...]
            outs[bi][...] = w * c

    in_specs = []
    for h in hdims:
        in_specs += [
            pl.BlockSpec((NGP, h), lambda i: (0, 0)),
            pl.BlockSpec((1, h), lambda i: (0, 0)),
            pl.BlockSpec((h, h), lambda i: (0, 0)),
            pl.BlockSpec((1, h), lambda i: (0, 0)),
        ]
    return pl.pallas_call(
        body,
        grid=(grid,),
        in_specs=in_specs,
        out_specs=tuple(pl.BlockSpec((tm, h), lambda i: (i, 0))
                        for h in hdims),
        out_shape=tuple(jax.ShapeDtypeStruct((M_TAB, h), _f32)
                        for h in hdims),
    )


@functools.lru_cache(maxsize=None)
def _node_update(h, nxt):
    r = 1000
    grid = N_NODES // r

    def body(a0_r, a1_r, h_r, l2w_r, l2b_r, lw_r, lb_r, *rest):
        agg = a0_r[...] + a1_r[...]
        x = _ssp(jnp.dot(agg, l2w_r[...], preferred_element_type=_f32)
                 + l2b_r[...])
        x = jnp.dot(x, lw_r[...], preferred_element_type=_f32) + lb_r[...]
        hn = h_r[...] + x
        if nxt:
            nw_r, hn_r, hlin_r = rest
            hn_r[...] = hn
            hlin_r[...] = jnp.dot(hn, nw_r[...], preferred_element_type=_f32)
        else:
            (hn_r,) = rest
            hn_r[...] = hn

    in_specs = [
        pl.BlockSpec((r, h), lambda i: (i, 0)),
        pl.BlockSpec((r, h), lambda i: (i, 0)),
        pl.BlockSpec((r, h), lambda i: (i, 0)),
        pl.BlockSpec((h, h), lambda i: (0, 0)),
        pl.BlockSpec((1, h), lambda i: (0, 0)),
        pl.BlockSpec((h, h), lambda i: (0, 0)),
        pl.BlockSpec((1, h), lambda i: (0, 0)),
    ]
    if nxt:
        in_specs.append(pl.BlockSpec((h, h), lambda i: (0, 0)))
        out_specs = (pl.BlockSpec((r, h), lambda i: (i, 0)),
                     pl.BlockSpec((r, h), lambda i: (i, 0)))
        out_shape = (jax.ShapeDtypeStruct((N_NODES, h), _f32),
                     jax.ShapeDtypeStruct((N_NODES, h), _f32))
    else:
        out_specs = pl.BlockSpec((r, h), lambda i: (i, 0))
        out_shape = jax.ShapeDtypeStruct((N_NODES, h), _f32)

    return pl.pallas_call(
        body,
        grid=(grid,),
        in_specs=in_specs,
        out_specs=out_specs,
        out_shape=out_shape,
    )


@functools.lru_cache(maxsize=None)
def _readout(h, with_prev):
    r = 1000
    grid = N_NODES // r
    hh = h // 2

    def body(h_r, w1_r, b1_r, w2_r, b2_r, bt_r, aux_r, out_r):
        i = pl.program_id(0)
        t = _ssp(jnp.dot(h_r[...], w1_r[...], preferred_element_type=_f32)
                 + b1_r[...])
        e = jnp.dot(t, w2_r[...], preferred_element_type=_f32) + b2_r[...]
        oh = (bt_r[...] == lax.broadcasted_iota(jnp.int32, (r, N_MOL), 1)
              ).astype(_f32)
        part = jnp.sum(oh * e, axis=0, keepdims=True)
        if with_prev:
            @pl.when(i == 0)
            def _():
                out_r[...] = aux_r[...]
            out_r[...] += part
        else:
            @pl.when(i == 0)
            def _():
                out_r[...] = jnp.zeros((1, N_MOL), _f32)
            out_r[...] += part * aux_r[0, 0]

    aux_spec = (pl.BlockSpec((1, N_MOL), lambda i: (0, 0)) if with_prev
                else pl.BlockSpec((1, 1), lambda i: (0, 0)))
    return pl.pallas_call(
        body,
        grid=(grid,),
        in_specs=[
            pl.BlockSpec((r, h), lambda i: (i, 0)),
            pl.BlockSpec((h, hh), lambda i: (0, 0)),
            pl.BlockSpec((1, hh), lambda i: (0, 0)),
            pl.BlockSpec((hh, 1), lambda i: (0, 0)),
            pl.BlockSpec((1, 1), lambda i: (0, 0)),
            pl.BlockSpec((r, 1), lambda i: (i, 0)),
            aux_spec,
        ],
        out_specs=pl.BlockSpec((1, N_MOL), lambda i: (0, 0)),
        out_shape=jax.ShapeDtypeStruct((1, N_MOL), _f32),
    )


def kernel(z, pos, edge_index, batch, low_params, dif_params, corr_w):
    src = edge_index[0].astype(jnp.int32)
    dst = edge_index[1].astype(jnp.int32)
    idx2 = jnp.concatenate([src, dst])
    pos16 = jnp.pad(pos.astype(_f32), ((0, 0), (0, 13)))
    pxy = _sc_gather(N_NODES, 16, 2 * N_EDGES, 2000)(pos16, idx2)
    px, py = pxy[:N_EDGES], pxy[N_EDGES:]
    z2 = z.reshape(-1, 1).astype(jnp.int32)
    b2 = batch.reshape(-1, 1).astype(jnp.int32)

    dall = _edge_d()(px, py).reshape(N_EDGES)

    all_blocks = low_params["blocks"] + dif_params["blocks"]
    hdims = tuple(blk["mlp_w2"].shape[0] for blk in all_blocks)
    tab_in = []
    for blk in all_blocks:
        tab_in += [jnp.pad(blk["mlp_w1"], ((0, NGP - NG), (0, 0))),
                   blk["mlp_b1"].reshape(1, -1), blk["mlp_w2"],
                   blk["mlp_b2"].reshape(1, -1)]
    tabs = _tables(hdims)(*tab_in)

    models = {"lo": (low_params, 128, tabs[:3]),
              "df": (dif_params, 64, tabs[3:])}
    state = {}
    for m, (params, hdim, _) in models.items():
        emb = jnp.pad(params["emb"], ((0, 4), (0, 0)))
        state[m] = _embed(hdim)(z2, emb, params["blocks"][0]["lin1_w"])

    seq = [("lo", 0), ("df", 0), ("lo", 1), ("df", 1), ("lo", 2)]
    h_out = {}
    for m, t in seq:
        params, hdim, tlist = models[m]
        blocks = params["blocks"]
        h, hlin = state[m]
        agg = _sc_msg_scatter(N_NODES, hdim, N_EDGES, 80)(
            tlist[t], dall, hlin, src, dst)
        blk = blocks[t]
        args = (agg[0], agg[1], h, blk["lin2_w"],
                blk["lin2_b"].reshape(1, -1), blk["lin_w"],
                blk["lin_b"].reshape(1, -1))
        if t + 1 < len(blocks):
            state[m] = _node_update(hdim, True)(*args,
                                                blocks[t + 1]["lin1_w"])
        else:
            h_out[m] = _node_update(hdim, False)(*args)

    h_low, h_dif = h_out["lo"], h_out["df"]

    y0 = _readout(128, False)(h_low, low_params["out1_w"],
                              low_params["out1_b"].reshape(1, -1),
                              low_params["out2_w"],
                              low_params["out2_b"].reshape(1, -1),
                              b2, corr_w)
    y = _readout(64, True)(h_dif, dif_params["out1_w"],
                           dif_params["out1_b"].reshape(1, -1),
                           dif_params["out2_w"],
                           dif_params["out2_b"].reshape(1, -1),
                           b2, y0)
    return y.reshape(N_MOL)


# speedup vs baseline: 4.1368x; 4.1368x over previous
import functools
import math

import jax
import jax.numpy as jnp
from jax import lax
from jax.experimental import pallas as pl
from jax.experimental.pallas import tpu as pltpu
from jax.experimental.pallas import tpu_sc as plsc

N_NODES = 10000
N_EDGES = 320000
N_MOL = 512
CUTOFF = 6.0
NG = 50
NGP = 64
NC = 2
NS = 16
NW = NC * NS

_f32 = jnp.float32


def _ssp(x):
    return jnp.maximum(x, 0.0) + jnp.log(1.0 + jnp.exp(-jnp.abs(x))) - math.log(2.0)


def _mesh():
    return plsc.VectorSubcoreMesh(core_axis_name="c", subcore_axis_name="s")


@functools.lru_cache(maxsize=None)
def _sc_gather(n_rows, d, n_idx, k):
    ew = n_idx // NW
    steps = ew // k
    assert ew % k == 0 and k % 8 == 0 and ew * NW == n_idx

    def body(table_hbm, idx_hbm, out_hbm, idx_v, rows_v, sem):
        wid = lax.axis_index("s") * NC + lax.axis_index("c")
        base = wid * ew

        def step(i, c):
            off = base + i * k
            pltpu.sync_copy(idx_hbm.at[pl.ds(off, k)], idx_v)
            pltpu.async_copy(table_hbm.at[idx_v], rows_v, sem).wait()
            pltpu.sync_copy(rows_v, out_hbm.at[pl.ds(off, k)])
            return c

        lax.fori_loop(0, steps, step, 0)

    return pl.kernel(
        body,
        out_type=jax.ShapeDtypeStruct((n_idx, d), _f32),
        mesh=_mesh(),
        compiler_params=pltpu.CompilerParams(use_tc_tiling_on_sc=False),
        scratch_types=[
            pltpu.VMEM((k,), jnp.int32),
            pltpu.VMEM((k, d), _f32),
            pltpu.SemaphoreType.DMA,
        ],
    )


@functools.lru_cache(maxsize=None)
def _sc_msg_scatter(n_rows, d, n_idx, k):
    ew = n_idx // NW
    steps = ew // k
    zr = n_rows // NS
    zc = 25
    zsteps = zr // zc
    assert ew % k == 0 and k % 8 == 0 and zr * NS == n_rows and zr % zc == 0
    assert steps % 2 == 1

    def body(w_hbm, hlin_hbm, src_hbm, dst_hbm, out_hbm,
             sidx0, sidx1, didx0, didx1, x0, x1, wr0, wr1, zbuf, accum,
             isem0, isem1, jsem0, jsem1, gsem0, gsem1, wsem0, wsem1):
        sidx = [sidx0, sidx1]
        didx = [didx0, didx1]
        xrows = [x0, x1]
        wrows = [wr0, wr1]
        isem = [isem0, isem1]
        jsem = [jsem0, jsem1]
        gsem = [gsem0, gsem1]
        wsem = [wsem0, wsem1]
        cid = lax.axis_index("c")
        sid = lax.axis_index("s")
        wid = sid * NC + cid
        base = wid * ew

        def zrow(r, c):
            for j in range(d // 16):
                zbuf[r, pl.ds(j * 16, 16)] = jnp.zeros((16,), _f32)
            return c

        lax.fori_loop(0, zc, zrow, 0)
        for j in range(zsteps):
            pltpu.sync_copy(zbuf, accum.at[pl.ds(sid * zr + j * zc, zc)])
        plsc.subcore_barrier()

        def eidx(i, b):
            off = base + i * k
            pltpu.async_copy(src_hbm.at[pl.ds(off, k)], sidx[b], isem[b])
            pltpu.async_copy(dst_hbm.at[pl.ds(off, k)], didx[b], jsem[b])

        def emain(i, b):
            off = base + i * k
            pltpu.make_async_copy(src_hbm.at[pl.ds(off, k)], sidx[b],
                                  isem[b]).wait()
            pltpu.async_copy(hlin_hbm.at[sidx[b]], xrows[b], gsem[b])
            pltpu.async_copy(w_hbm.at[pl.ds(off, k)], wrows[b], wsem[b])

        def consume(i, b):
            off = base + i * k
            pltpu.make_async_copy(hlin_hbm.at[sidx[b]], xrows[b],
                                  gsem[b]).wait()
            pltpu.make_async_copy(w_hbm.at[pl.ds(off, k)], wrows[b],
                                  wsem[b]).wait()
            pltpu.make_async_copy(dst_hbm.at[pl.ds(off, k)], didx[b],
                                  jsem[b]).wait()

            def mrow(r, c2):
                for j in range(d // 16):
                    sl = pl.ds(j * 16, 16)
                    wrows[b][r, sl] = wrows[b][r, sl] * xrows[b][r, sl]
                return c2

            lax.fori_loop(0, k, mrow, 0)
            pltpu.sync_copy(wrows[b], accum.at[didx[b]], add=True)

        eidx(0, 0)
        eidx(1, 1)
        emain(0, 0)

        def pair(p, c):
            i0 = 2 * p
            emain(i0 + 1, 1)
            consume(i0, 0)
            eidx(i0 + 2, 0)
            i1 = i0 + 1
            emain(i1 + 1, 0)
            consume(i1, 1)

            @pl.when(i1 + 2 < steps)
            def _():
                eidx(i1 + 2, 1)

            return c

        lax.fori_loop(0, (steps - 1) // 2, pair, 0)
        consume(steps - 1, 0)
        plsc.subcore_barrier()

        pltpu.sync_copy(accum.at[pl.ds(sid * zr, zr)],
                        out_hbm.at[cid].at[pl.ds(sid * zr, zr)])

    return pl.kernel(
        body,
        out_type=jax.ShapeDtypeStruct((NC, n_rows, d), _f32),
        mesh=_mesh(),
        compiler_params=pltpu.CompilerParams(use_tc_tiling_on_sc=False),
        scratch_types=[
            pltpu.VMEM((k,), jnp.int32),
            pltpu.VMEM((k,), jnp.int32),
            pltpu.VMEM((k,), jnp.int32),
            pltpu.VMEM((k,), jnp.int32),
            pltpu.VMEM((k, d), _f32),
            pltpu.VMEM((k, d), _f32),
            pltpu.VMEM((k, d), _f32),
            pltpu.VMEM((k, d), _f32),
            pltpu.VMEM((zc, d), _f32),
            pltpu.VMEM_SHARED((n_rows, d), _f32),
            pltpu.SemaphoreType.DMA,
            pltpu.SemaphoreType.DMA,
            pltpu.SemaphoreType.DMA,
            pltpu.SemaphoreType.DMA,
            pltpu.SemaphoreType.DMA,
            pltpu.SemaphoreType.DMA,
            pltpu.SemaphoreType.DMA,
            pltpu.SemaphoreType.DMA,
        ],
    )


@functools.lru_cache(maxsize=None)
def _embed(h):
    r = 1000
    grid = N_NODES // r
    zp = 104

    def body(z_r, emb_r, l1_r, h_r, hlin_r):
        oh = (z_r[...] == lax.broadcasted_iota(jnp.int32, (r, zp), 1)
              ).astype(_f32)
        h0 = jnp.dot(oh, emb_r[...], preferred_element_type=_f32)
        h_r[...] = h0
        hlin_r[...] = jnp.dot(h0, l1_r[...], preferred_element_type=_f32)

    return pl.pallas_call(
        body,
        grid=(grid,),
        in_specs=[
            pl.BlockSpec((r, 1), lambda i: (i, 0)),
            pl.BlockSpec((zp, h), lambda i: (0, 0)),
            pl.BlockSpec((h, h), lambda i: (0, 0)),
        ],
        out_specs=(pl.BlockSpec((r, h), lambda i: (i, 0)),
                   pl.BlockSpec((r, h), lambda i: (i, 0))),
        out_shape=(jax.ShapeDtypeStruct((N_NODES, h), _f32),
                   jax.ShapeDtypeStruct((N_NODES, h), _f32)),
    )


@functools.lru_cache(maxsize=None)
def _edge_wall(hdims):
    te = 2000
    grid = N_EDGES // te
    delta = CUTOFF / (NG - 1)
    coeff = -0.5 / delta ** 2
    nb = len(hdims)

    def body(*refs):
        px_r, py_r = refs[0], refs[1]
        ins = refs[2:2 + 4 * nb]
        outs = refs[2 + 4 * nb:]
        dxyz = px_r[...] - py_r[...]
        d2 = jnp.sum(dxyz * dxyz, axis=1, keepdims=True)
        d = jnp.sqrt(d2 + 1e-12)
        offs = lax.broadcasted_iota(jnp.int32, (te, NGP), 1).astype(_f32) * delta
        dd = d - offs
        rbf = jnp.exp(coeff * (dd * dd))
        c = 0.5 * (jnp.cos(d * (math.pi / CUTOFF)) + 1.0)
        c = jnp.where(d < CUTOFF, c, 0.0)
        for bi in range(nb):
            w1_r, b1_r, w2_r, b2_r = ins[4 * bi:4 * bi + 4]
            t = _ssp(jnp.dot(rbf, w1_r[...], preferred_element_type=_f32)
                     + b1_r[...])
            w = jnp.dot(t, w2_r[...], preferred_element_type=_f32) + b2_r[...]
            outs[bi][...] = w * c

    in_specs = [pl.BlockSpec((te, 16), lambda i: (i, 0)),
                pl.BlockSpec((te, 16), lambda i: (i, 0))]
    for h in hdims:
        in_specs += [
            pl.BlockSpec((NGP, h), lambda i: (0, 0)),
            pl.BlockSpec((1, h), lambda i: (0, 0)),
            pl.BlockSpec((h, h), lambda i: (0, 0)),
            pl.BlockSpec((1, h), lambda i: (0, 0)),
        ]
    return pl.pallas_call(
        body,
        grid=(grid,),
        in_specs=in_specs,
        out_specs=tuple(pl.BlockSpec((te, h), lambda i: (i, 0))
                        for h in hdims),
        out_shape=tuple(jax.ShapeDtypeStruct((N_EDGES, h), _f32)
                        for h in hdims),
    )


@functools.lru_cache(maxsize=None)
def _node_update(h, nxt):
    r = 1000
    grid = N_NODES // r

    def body(a0_r, a1_r, h_r, l2w_r, l2b_r, lw_r, lb_r, *rest):
        agg = a0_r[...] + a1_r[...]
        x = _ssp(jnp.dot(agg, l2w_r[...], preferred_element_type=_f32)
                 + l2b_r[...])
        x = jnp.dot(x, lw_r[...], preferred_element_type=_f32) + lb_r[...]
        hn = h_r[...] + x
        if nxt:
            nw_r, hn_r, hlin_r = rest
            hn_r[...] = hn
            hlin_r[...] = jnp.dot(hn, nw_r[...], preferred_element_type=_f32)
        else:
            (hn_r,) = rest
            hn_r[...] = hn

    in_specs = [
        pl.BlockSpec((r, h), lambda i: (i, 0)),
        pl.BlockSpec((r, h), lambda i: (i, 0)),
        pl.BlockSpec((r, h), lambda i: (i, 0)),
        pl.BlockSpec((h, h), lambda i: (0, 0)),
        pl.BlockSpec((1, h), lambda i: (0, 0)),
        pl.BlockSpec((h, h), lambda i: (0, 0)),
        pl.BlockSpec((1, h), lambda i: (0, 0)),
    ]
    if nxt:
        in_specs.append(pl.BlockSpec((h, h), lambda i: (0, 0)))
        out_specs = (pl.BlockSpec((r, h), lambda i: (i, 0)),
                     pl.BlockSpec((r, h), lambda i: (i, 0)))
        out_shape = (jax.ShapeDtypeStruct((N_NODES, h), _f32),
                     jax.ShapeDtypeStruct((N_NODES, h), _f32))
    else:
        out_specs = pl.BlockSpec((r, h), lambda i: (i, 0))
        out_shape = jax.ShapeDtypeStruct((N_NODES, h), _f32)

    return pl.pallas_call(
        body,
        grid=(grid,),
        in_specs=in_specs,
        out_specs=out_specs,
        out_shape=out_shape,
    )


@functools.lru_cache(maxsize=None)
def _readout(h, with_prev):
    r = 1000
    grid = N_NODES // r
    hh = h // 2

    def body(h_r, w1_r, b1_r, w2_r, b2_r, bt_r, aux_r, out_r):
        i = pl.program_id(0)
        t = _ssp(jnp.dot(h_r[...], w1_r[...], preferred_element_type=_f32)
                 + b1_r[...])
        e = jnp.dot(t, w2_r[...], preferred_element_type=_f32) + b2_r[...]
        oh = (bt_r[...] == lax.broadcasted_iota(jnp.int32, (r, N_MOL), 1)
              ).astype(_f32)
        part = jnp.sum(oh * e, axis=0, keepdims=True)
        if with_prev:
            @pl.when(i == 0)
            def _():
                out_r[...] = aux_r[...]
            out_r[...] += part
        else:
            @pl.when(i == 0)
            def _():
                out_r[...] = jnp.zeros((1, N_MOL), _f32)
            out_r[...] += part * aux_r[0, 0]

    aux_spec = (pl.BlockSpec((1, N_MOL), lambda i: (0, 0)) if with_prev
                else pl.BlockSpec((1, 1), lambda i: (0, 0)))
    return pl.pallas_call(
        body,
        grid=(grid,),
        in_specs=[
            pl.BlockSpec((r, h), lambda i: (i, 0)),
            pl.BlockSpec((h, hh), lambda i: (0, 0)),
            pl.BlockSpec((1, hh), lambda i: (0, 0)),
            pl.BlockSpec((hh, 1), lambda i: (0, 0)),
            pl.BlockSpec((1, 1), lambda i: (0, 0)),
            pl.BlockSpec((r, 1), lambda i: (i, 0)),
            aux_spec,
        ],
        out_specs=pl.BlockSpec((1, N_MOL), lambda i: (0, 0)),
        out_shape=jax.ShapeDtypeStruct((1, N_MOL), _f32),
    )


def kernel(z, pos, edge_index, batch, low_params, dif_params, corr_w):
    src = edge_index[0].astype(jnp.int32)
    dst = edge_index[1].astype(jnp.int32)
    idx2 = jnp.concatenate([src, dst])
    pos16 = jnp.pad(pos.astype(_f32), ((0, 0), (0, 13)))
    pxy = _sc_gather(N_NODES, 16, 2 * N_EDGES, 2000)(pos16, idx2)
    px, py = pxy[:N_EDGES], pxy[N_EDGES:]
    z2 = z.reshape(-1, 1).astype(jnp.int32)
    b2 = batch.reshape(-1, 1).astype(jnp.int32)

    all_blocks = low_params["blocks"] + dif_params["blocks"]
    hdims = tuple(blk["mlp_w2"].shape[0] for blk in all_blocks)
    wall_in = []
    for blk in all_blocks:
        wall_in += [jnp.pad(blk["mlp_w1"], ((0, NGP - NG), (0, 0))),
                    blk["mlp_b1"].reshape(1, -1), blk["mlp_w2"],
                    blk["mlp_b2"].reshape(1, -1)]
    ws = _edge_wall(hdims)(px, py, *wall_in)

    models = {"lo": (low_params, 128, ws[:3]), "df": (dif_params, 64, ws[3:])}
    state = {}
    for m, (params, hdim, _) in models.items():
        emb = jnp.pad(params["emb"], ((0, 4), (0, 0)))
        state[m] = _embed(hdim)(z2, emb, params["blocks"][0]["lin1_w"])

    seq = [("lo", 0), ("df", 0), ("lo", 1), ("df", 1), ("lo", 2)]
    h_out = {}
    for m, t in seq:
        params, hdim, wlist = models[m]
        blocks = params["blocks"]
        h, hlin = state[m]
        agg = _sc_msg_scatter(N_NODES, hdim, N_EDGES, 80)(
            wlist[t], hlin, src, dst)
        blk = blocks[t]
        args = (agg[0], agg[1], h, blk["lin2_w"],
                blk["lin2_b"].reshape(1, -1), blk["lin_w"],
                blk["lin_b"].reshape(1, -1))
        if t + 1 < len(blocks):
            state[m] = _node_update(hdim, True)(*args,
                                                blocks[t + 1]["lin1_w"])
        else:
            h_out[m] = _node_update(hdim, False)(*args)

    h_low, h_dif = h_out["lo"], h_out["df"]

    y0 = _readout(128, False)(h_low, low_params["out1_w"],
                              low_params["out1_b"].reshape(1, -1),
                              low_params["out2_w"],
                              low_params["out2_b"].reshape(1, -1),
                              b2, corr_w)
    y = _readout(64, True)(h_dif, dif_params["out1_w"],
                           dif_params["out1_b"].reshape(1, -1),
                           dif_params["out2_w"],
                           dif_params["out2_b"].reshape(1, -1),
                           b2, y0)
    return y.reshape(N_MOL)
